# baseline (speedup 1.0000x reference)
import jax
import jax.numpy as jnp
from jax.experimental import pallas as pl

_BBLK = 8
_K = 64


def _fused_body(ne_ref, hs_ref, w_ref, b_ref,
                emb_ref, pol_ref, scr_ref, ent_ref, idx_ref):
    f32 = jnp.float32
    hs = hs_ref[...]
    W = w_ref[...]
    bb = b_ref[...]

    scorer = jnp.tanh(
        jax.lax.dot_general(hs, W, (((1,), (1,)), ((), ())),
                            preferred_element_type=f32) + bb)
    scr_ref[...] = scorer
    norm = jnp.sqrt(jnp.sum(scorer * scorer, axis=1, keepdims=True))

    rows = []
    for i in range(_BBLK):
        s_i = jax.lax.dot_general(scorer[i:i + 1], ne_ref[i],
                                  (((1,), (1,)), ((), ())),
                                  preferred_element_type=f32)
        rows.append(s_i)
    scores = jnp.concatenate(rows, axis=0) / norm

    G = scores.shape[1]

    NCH = 4
    CW = G // NCH
    cand_vals, cand_idxs = [], []
    iota_c = jax.lax.broadcasted_iota(jnp.int32, (_BBLK, CW), 1)
    sub_vals = [[] for _ in range(NCH)]
    sub_idxs = [[] for _ in range(NCH)]
    works = [scores[:, c * CW:(c + 1) * CW] for c in range(NCH)]
    for _ in range(_K):
        for c in range(NCH):
            m = jnp.max(works[c], axis=1, keepdims=True)
            cnd = jnp.where(works[c] == m, iota_c, CW)
            ik = jnp.min(cnd, axis=1, keepdims=True)
            sub_vals[c].append(m)
            sub_idxs[c].append(ik + c * CW)
            works[c] = jnp.where(iota_c == ik, -jnp.inf, works[c])
    for c in range(NCH):
        cand_vals.append(jnp.concatenate(sub_vals[c], axis=1))
        cand_idxs.append(jnp.concatenate(sub_idxs[c], axis=1))
    cv = jnp.concatenate(cand_vals, axis=1)
    ci = jnp.concatenate(cand_idxs, axis=1)

    vals_l, idx_l = [], []
    for _ in range(_K):
        m = jnp.max(cv, axis=1, keepdims=True)
        cnd = jnp.where(cv == m, ci, G)
        ik = jnp.min(cnd, axis=1, keepdims=True)
        vals_l.append(m)
        idx_l.append(ik)
        cv = jnp.where((cv == m) & (ci == ik), -jnp.inf, cv)
    vals = jnp.concatenate(vals_l, axis=1)
    idxs = jnp.concatenate(idx_l, axis=1)

    m0 = vals[:, 0:1]
    e = jnp.exp(scores - m0)
    z = jnp.sum(e, axis=1, keepdims=True)
    logz = m0 + jnp.log(z)
    ps = jnp.sum(e * scores, axis=1, keepdims=True) / z
    ent_ref[...] = logz - ps
    pol_ref[...] = jnp.mean(vals, axis=1, keepdims=True) - logz

    reps = idx_ref.shape[1] // _K
    idx_ref[...] = jnp.concatenate([idxs] * reps, axis=1)

    tanh_vals = jnp.tanh(vals)
    iota_s = jax.lax.broadcasted_iota(jnp.int32, (G, _K), 0)
    for i in range(_BBLK):
        oh = (iota_s == idxs[i:i + 1, :]).astype(f32)
        g_t = jax.lax.dot_general(ne_ref[i], oh,
                                  (((0,), (0,)), ((), ())),
                                  preferred_element_type=f32)
        g_t = g_t * tanh_vals[i:i + 1, :]
        emb_ref[i] = jnp.concatenate([g_t] * reps, axis=1)


def kernel(node_embs, mask, h_selector, W, b):
    del mask
    B, G, D = node_embs.shape
    RNN = h_selector.shape[1]
    b2 = b.reshape(1, D)
    nblk = B // _BBLK

    out_shape = (
        jax.ShapeDtypeStruct((B, D, D), jnp.float32),
        jax.ShapeDtypeStruct((B, 1), jnp.float32),
        jax.ShapeDtypeStruct((B, D), jnp.float32),
        jax.ShapeDtypeStruct((B, 1), jnp.float32),
        jax.ShapeDtypeStruct((B, D), jnp.int32),
    )
    emb, pol, scr, ent, idx = pl.pallas_call(
        _fused_body,
        grid=(nblk,),
        in_specs=[
            pl.BlockSpec((_BBLK, G, D), lambda i: (i, 0, 0)),
            pl.BlockSpec((_BBLK, RNN), lambda i: (i, 0)),
            pl.BlockSpec((D, RNN), lambda i: (0, 0)),
            pl.BlockSpec((1, D), lambda i: (0, 0)),
        ],
        out_specs=[
            pl.BlockSpec((_BBLK, D, D), lambda i: (i, 0, 0)),
            pl.BlockSpec((_BBLK, 1), lambda i: (i, 0)),
            pl.BlockSpec((_BBLK, D), lambda i: (i, 0)),
            pl.BlockSpec((_BBLK, 1), lambda i: (i, 0)),
            pl.BlockSpec((_BBLK, D), lambda i: (i, 0)),
        ],
        out_shape=out_shape,
    )(node_embs, h_selector, W, b2)
    return emb, pol[:, 0], scr, ent[:, 0], idx


# speedup vs baseline: 2.4628x; 2.4628x over previous
import jax
import jax.numpy as jnp
from jax.experimental import pallas as pl
from jax.experimental.pallas import tpu as pltpu

_BBLK = 8
_K = 64


def _topk_bitonic(scores):
    R, G = scores.shape
    lane = jax.lax.broadcasted_iota(jnp.int32, (R, G), 1)
    v = scores
    ix = lane

    def roll(x, s):
        return pltpu.roll(x, s % G, 1)

    def stage(v, ix, d, dirmask):
        lowbit = (lane & d) == 0
        vp = jnp.where(lowbit, roll(v, -d), roll(v, d))
        ip = jnp.where(lowbit, roll(ix, -d), roll(ix, d))
        mine_wins = (v > vp) | ((v == vp) & (ix < ip))
        keep = mine_wins == (lowbit == dirmask)
        return jnp.where(keep, v, vp), jnp.where(keep, ix, ip)

    for bs_log in range(1, 7):
        dirmask = (lane & (1 << bs_log)) == 0
        for d_log in reversed(range(bs_log)):
            v, ix = stage(v, ix, 1 << d_log, dirmask)
    all_true = lane >= 0
    for lvl in range(5):
        v, ix = stage(v, ix, 64 << lvl, all_true)
        dirmask = (lane & (128 << lvl)) == 0
        for d_log in reversed(range(6)):
            v, ix = stage(v, ix, 1 << d_log, dirmask)
    return v[:, :_K], ix[:, :_K]


def _fused_body(ne_ref, hs_ref, w_ref, b_ref,
                emb_ref, pol_ref, scr_ref, ent_ref, idx_ref):
    f32 = jnp.float32
    hs = hs_ref[...]
    W = w_ref[...]
    bb = b_ref[...]

    scorer = jnp.tanh(
        jax.lax.dot_general(hs, W, (((1,), (1,)), ((), ())),
                            preferred_element_type=f32) + bb)
    scr_ref[...] = scorer
    norm = jnp.sqrt(jnp.sum(scorer * scorer, axis=1, keepdims=True))

    rows = []
    for i in range(_BBLK):
        s_i = jax.lax.dot_general(scorer[i:i + 1], ne_ref[i],
                                  (((1,), (1,)), ((), ())),
                                  preferred_element_type=f32)
        rows.append(s_i)
    scores = jnp.concatenate(rows, axis=0) / norm

    G = scores.shape[1]
    vals, idxs = _topk_bitonic(scores)

    m0 = vals[:, 0:1]
    e = jnp.exp(scores - m0)
    z = jnp.sum(e, axis=1, keepdims=True)
    logz = m0 + jnp.log(z)
    ps = jnp.sum(e * scores, axis=1, keepdims=True) / z
    ent_ref[...] = logz - ps
    pol_ref[...] = jnp.mean(vals, axis=1, keepdims=True) - logz

    reps = idx_ref.shape[1] // _K
    idx_ref[...] = jnp.concatenate([idxs] * reps, axis=1)

    tanh_vals = jnp.tanh(vals)
    iota_s = jax.lax.broadcasted_iota(jnp.int32, (G, _K), 0)
    for i in range(_BBLK):
        oh = (iota_s == idxs[i:i + 1, :]).astype(f32)
        g_t = jax.lax.dot_general(ne_ref[i], oh,
                                  (((0,), (0,)), ((), ())),
                                  preferred_element_type=f32)
        g_t = g_t * tanh_vals[i:i + 1, :]
        emb_ref[i] = jnp.concatenate([g_t] * reps, axis=1)


def kernel(node_embs, mask, h_selector, W, b):
    del mask
    B, G, D = node_embs.shape
    RNN = h_selector.shape[1]
    b2 = b.reshape(1, D)
    nblk = B // _BBLK

    out_shape = (
        jax.ShapeDtypeStruct((B, D, D), jnp.float32),
        jax.ShapeDtypeStruct((B, 1), jnp.float32),
        jax.ShapeDtypeStruct((B, D), jnp.float32),
        jax.ShapeDtypeStruct((B, 1), jnp.float32),
        jax.ShapeDtypeStruct((B, D), jnp.int32),
    )
    emb, pol, scr, ent, idx = pl.pallas_call(
        _fused_body,
        grid=(nblk,),
        in_specs=[
            pl.BlockSpec((_BBLK, G, D), lambda i: (i, 0, 0)),
            pl.BlockSpec((_BBLK, RNN), lambda i: (i, 0)),
            pl.BlockSpec((D, RNN), lambda i: (0, 0)),
            pl.BlockSpec((1, D), lambda i: (0, 0)),
        ],
        out_specs=[
            pl.BlockSpec((_BBLK, D, D), lambda i: (i, 0, 0)),
            pl.BlockSpec((_BBLK, 1), lambda i: (i, 0)),
            pl.BlockSpec((_BBLK, D), lambda i: (i, 0)),
            pl.BlockSpec((_BBLK, 1), lambda i: (i, 0)),
            pl.BlockSpec((_BBLK, D), lambda i: (i, 0)),
        ],
        out_shape=out_shape,
    )(node_embs, h_selector, W, b2)
    return emb, pol[:, 0], scr, ent[:, 0], idx


# speedup vs baseline: 2.6379x; 1.0711x over previous
import jax
import jax.numpy as jnp
from jax.experimental import pallas as pl
from jax.experimental.pallas import tpu as pltpu

_BBLK = 8
_K = 64


def _topk_bitonic(scores):
    R, G = scores.shape
    lane_g = jax.lax.broadcasted_iota(jnp.int32, (R, G), 1)
    v = scores
    ix = lane_g

    def stage(v, ix, d, dirmask, lane):
        W = v.shape[1]
        lowbit = (lane & d) == 0
        vp = jnp.where(lowbit, pltpu.roll(v, (-d) % W, 1), pltpu.roll(v, d, 1))
        ip = jnp.where(lowbit, pltpu.roll(ix, (-d) % W, 1), pltpu.roll(ix, d, 1))
        mine_wins = (v > vp) | ((v == vp) & (ix < ip))
        keep = mine_wins == (lowbit == dirmask)
        return jnp.where(keep, v, vp), jnp.where(keep, ix, ip)

    for bs_log in range(1, 7):
        dirmask = (lane_g & (1 << bs_log)) == 0
        for d_log in reversed(range(bs_log)):
            v, ix = stage(v, ix, 1 << d_log, dirmask, lane_g)
    while True:
        W = v.shape[1]
        lane = lane_g[:, :W]
        v, ix = stage(v, ix, 64, lane >= 0, lane)
        dirmask = (lane & 128) == 0
        for d_log in reversed(range(6)):
            v, ix = stage(v, ix, 1 << d_log, dirmask, lane)
        if W == 128:
            break
        v = jnp.concatenate([v[:, m * 128:m * 128 + 64]
                             for m in range(W // 128)], axis=1)
        ix = jnp.concatenate([ix[:, m * 128:m * 128 + 64]
                              for m in range(W // 128)], axis=1)
    return v[:, :_K], ix[:, :_K]


def _fused_body(ne_ref, hs_ref, w_ref, b_ref,
                emb_ref, pol_ref, scr_ref, ent_ref, idx_ref):
    f32 = jnp.float32
    hs = hs_ref[...]
    W = w_ref[...]
    bb = b_ref[...]

    scorer = jnp.tanh(
        jax.lax.dot_general(hs, W, (((1,), (1,)), ((), ())),
                            preferred_element_type=f32) + bb)
    scr_ref[...] = scorer
    norm = jnp.sqrt(jnp.sum(scorer * scorer, axis=1, keepdims=True))

    rows = []
    for i in range(_BBLK):
        s_i = jax.lax.dot_general(scorer[i:i + 1], ne_ref[i],
                                  (((1,), (1,)), ((), ())),
                                  preferred_element_type=f32)
        rows.append(s_i)
    scores = jnp.concatenate(rows, axis=0) / norm

    G = scores.shape[1]
    vals, idxs = _topk_bitonic(scores)

    m0 = vals[:, 0:1]
    e = jnp.exp(scores - m0)
    z = jnp.sum(e, axis=1, keepdims=True)
    logz = m0 + jnp.log(z)
    ps = jnp.sum(e * scores, axis=1, keepdims=True) / z
    ent_ref[...] = logz - ps
    pol_ref[...] = jnp.mean(vals, axis=1, keepdims=True) - logz

    reps = idx_ref.shape[1] // _K
    idx_ref[...] = jnp.concatenate([idxs] * reps, axis=1)

    tanh_vals = jnp.tanh(vals)
    iota_s = jax.lax.broadcasted_iota(jnp.int32, (G, _K), 0)
    for i in range(_BBLK):
        oh = (iota_s == idxs[i:i + 1, :]).astype(f32)
        g_t = jax.lax.dot_general(ne_ref[i], oh,
                                  (((0,), (0,)), ((), ())),
                                  preferred_element_type=f32)
        g_t = g_t * tanh_vals[i:i + 1, :]
        emb_ref[i] = jnp.concatenate([g_t] * reps, axis=1)


def kernel(node_embs, mask, h_selector, W, b):
    del mask
    B, G, D = node_embs.shape
    RNN = h_selector.shape[1]
    b2 = b.reshape(1, D)
    nblk = B // _BBLK

    out_shape = (
        jax.ShapeDtypeStruct((B, D, D), jnp.float32),
        jax.ShapeDtypeStruct((B, 1), jnp.float32),
        jax.ShapeDtypeStruct((B, D), jnp.float32),
        jax.ShapeDtypeStruct((B, 1), jnp.float32),
        jax.ShapeDtypeStruct((B, D), jnp.int32),
    )
    emb, pol, scr, ent, idx = pl.pallas_call(
        _fused_body,
        grid=(nblk,),
        in_specs=[
            pl.BlockSpec((_BBLK, G, D), lambda i: (i, 0, 0)),
            pl.BlockSpec((_BBLK, RNN), lambda i: (i, 0)),
            pl.BlockSpec((D, RNN), lambda i: (0, 0)),
            pl.BlockSpec((1, D), lambda i: (0, 0)),
        ],
        out_specs=[
            pl.BlockSpec((_BBLK, D, D), lambda i: (i, 0, 0)),
            pl.BlockSpec((_BBLK, 1), lambda i: (i, 0)),
            pl.BlockSpec((_BBLK, D), lambda i: (i, 0)),
            pl.BlockSpec((_BBLK, 1), lambda i: (i, 0)),
            pl.BlockSpec((_BBLK, D), lambda i: (i, 0)),
        ],
        out_shape=out_shape,
    )(node_embs, h_selector, W, b2)
    return emb, pol[:, 0], scr, ent[:, 0], idx


# speedup vs baseline: 2.7711x; 1.0505x over previous
import jax
import jax.numpy as jnp
from jax.experimental import pallas as pl
from jax.experimental.pallas import tpu as pltpu

_BBLK = 8
_K = 64


def _topk_bitonic(scores):
    R, G = scores.shape
    lane_g = jax.lax.broadcasted_iota(jnp.int32, (R, G), 1)
    v = scores
    ix = lane_g

    def stage(v, ix, d, dirmask, lane):
        W = v.shape[1]
        lowbit = (lane & d) == 0
        vp = jnp.where(lowbit, pltpu.roll(v, (-d) % W, 1), pltpu.roll(v, d, 1))
        ip = jnp.where(lowbit, pltpu.roll(ix, (-d) % W, 1), pltpu.roll(ix, d, 1))
        mine_wins = (v > vp) | ((v == vp) & (ix < ip))
        keep = mine_wins == (lowbit == dirmask)
        return jnp.where(keep, v, vp), jnp.where(keep, ix, ip)

    for bs_log in range(1, 7):
        dirmask = (lane_g & (1 << bs_log)) == 0
        for d_log in reversed(range(bs_log)):
            v, ix = stage(v, ix, 1 << d_log, dirmask, lane_g)
    while True:
        W = v.shape[1]
        lane = lane_g[:, :W]
        v, ix = stage(v, ix, 64, lane >= 0, lane)
        dirmask = (lane & 128) == 0
        for d_log in reversed(range(6)):
            v, ix = stage(v, ix, 1 << d_log, dirmask, lane)
        if W == 128:
            break
        v = jnp.concatenate([v[:, m * 128:m * 128 + 64]
                             for m in range(W // 128)], axis=1)
        ix = jnp.concatenate([ix[:, m * 128:m * 128 + 64]
                              for m in range(W // 128)], axis=1)
    return v[:, :_K], ix[:, :_K]


def _fused_body(ne_ref, hs_ref, w_ref, b_ref,
                emb_ref, pol_ref, scr_ref, ent_ref, idx_ref):
    f32 = jnp.float32
    hs = hs_ref[...]
    W = w_ref[...]
    bb = b_ref[...]

    scorer = jnp.tanh(
        jax.lax.dot_general(hs, W, (((1,), (1,)), ((), ())),
                            preferred_element_type=f32) + bb)
    scr_ref[...] = scorer
    norm = jnp.sqrt(jnp.sum(scorer * scorer, axis=1, keepdims=True))

    rows = []
    for i in range(_BBLK):
        s_i = jax.lax.dot_general(scorer[i:i + 1], ne_ref[i],
                                  (((1,), (1,)), ((), ())),
                                  preferred_element_type=f32)
        rows.append(s_i)
    scores = jnp.concatenate(rows, axis=0) / norm

    G = scores.shape[1]
    vals, idxs = _topk_bitonic(scores)

    m0 = vals[:, 0:1]
    e = jnp.exp(scores - m0)
    z = jnp.sum(e, axis=1, keepdims=True)
    logz = m0 + jnp.log(z)
    ps = jnp.sum(e * scores, axis=1, keepdims=True) / z
    ent_ref[...] = logz - ps
    pol_ref[...] = jnp.mean(vals, axis=1, keepdims=True) - logz

    reps = idx_ref.shape[1] // _K
    idx_ref[...] = jnp.concatenate([idxs] * reps, axis=1)

    tanh_vals = jnp.tanh(vals)
    idxs_t = jnp.transpose(idxs)
    iota_l = jax.lax.broadcasted_iota(jnp.int32, (_K, G), 1)
    for i in range(_BBLK):
        oh_t = (iota_l == idxs_t[:, i:i + 1]).astype(jnp.bfloat16)
        g = jax.lax.dot_general(oh_t, ne_ref[i],
                                (((1,), (0,)), ((), ())),
                                preferred_element_type=f32)
        g_t = jnp.transpose(g) * tanh_vals[i:i + 1, :]
        emb_ref[i] = jnp.concatenate([g_t] * reps, axis=1)


def kernel(node_embs, mask, h_selector, W, b):
    del mask
    B, G, D = node_embs.shape
    RNN = h_selector.shape[1]
    b2 = b.reshape(1, D)
    nblk = B // _BBLK

    out_shape = (
        jax.ShapeDtypeStruct((B, D, D), jnp.float32),
        jax.ShapeDtypeStruct((B, 1), jnp.float32),
        jax.ShapeDtypeStruct((B, D), jnp.float32),
        jax.ShapeDtypeStruct((B, 1), jnp.float32),
        jax.ShapeDtypeStruct((B, D), jnp.int32),
    )
    emb, pol, scr, ent, idx = pl.pallas_call(
        _fused_body,
        grid=(nblk,),
        in_specs=[
            pl.BlockSpec((_BBLK, G, D), lambda i: (i, 0, 0)),
            pl.BlockSpec((_BBLK, RNN), lambda i: (i, 0)),
            pl.BlockSpec((D, RNN), lambda i: (0, 0)),
            pl.BlockSpec((1, D), lambda i: (0, 0)),
        ],
        out_specs=[
            pl.BlockSpec((_BBLK, D, D), lambda i: (i, 0, 0)),
            pl.BlockSpec((_BBLK, 1), lambda i: (i, 0)),
            pl.BlockSpec((_BBLK, D), lambda i: (i, 0)),
            pl.BlockSpec((_BBLK, 1), lambda i: (i, 0)),
            pl.BlockSpec((_BBLK, D), lambda i: (i, 0)),
        ],
        out_shape=out_shape,
    )(node_embs, h_selector, W, b2)
    return emb, pol[:, 0], scr, ent[:, 0], idx
